# baseline (speedup 1.0000x reference)
import functools

import jax
import jax.numpy as jnp
from jax import lax
from jax.experimental import pallas as pl
from jax.experimental.pallas import tpu as pltpu
from jax.experimental.pallas import tpu_sc as plsc

D_VAR_ = 0.5
NLAB_PAD = 128
LANES = 16
NTILES = 32
BM = 8192


NBUF = 24
CHM = 2048


def _hinge_body(x_hbm, o_hbm, bufs, obuf, isem, osem):
    B, C, M = x_hbm.shape
    nchunks = B * (M // CHM)
    per_b = M // CHM

    def in_copy(i, s):
        b, mc = divmod(i, per_b)
        return pltpu.make_async_copy(
            x_hbm.at[b, :, pl.ds(mc * CHM, CHM)], bufs.at[s], isem.at[s])

    def out_copy(i, s):
        b, mc = divmod(i, per_b)
        return pltpu.make_async_copy(
            obuf.at[s], o_hbm.at[pl.ds(b, 1), pl.ds(mc * CHM, CHM)],
            osem.at[s])

    for j in range(NBUF):
        in_copy(j, j).start()

    for i in range(nchunks):
        s = i % NBUF
        in_copy(i, s).wait()
        x = bufs[s]
        s1 = jnp.sum(x, axis=0, keepdims=True)
        s2 = jnp.sum(x * x, axis=0, keepdims=True)
        var = jnp.maximum(s2 - s1 * s1 * (1.0 / C), 0.0)
        h = jnp.maximum(jnp.sqrt(var) - D_VAR_, 0.0)
        if i >= NBUF:
            out_copy(i - NBUF, s).wait()
        obuf[s] = h * h
        out_copy(i, s).start()
        nxt = i + NBUF
        if nxt < nchunks:
            in_copy(nxt, s).start()

    for i in range(nchunks - NBUF, nchunks):
        out_copy(i, i % NBUF).wait()


def _hinge(prediction):
    B, C, M = prediction.shape
    return pl.pallas_call(
        _hinge_body,
        in_specs=[pl.BlockSpec(memory_space=pl.ANY)],
        out_specs=pl.BlockSpec(memory_space=pl.ANY),
        out_shape=jax.ShapeDtypeStruct((B, M), jnp.float32),
        scratch_shapes=[
            pltpu.VMEM((NBUF, 32, CHM), jnp.float32),
            pltpu.VMEM((NBUF, 1, CHM), jnp.float32),
            pltpu.SemaphoreType.DMA((NBUF,)),
            pltpu.SemaphoreType.DMA((NBUF,)),
        ],
    )(prediction)


NSUB = 4
UNROLL = 8


def _tile_coords(chunk, B):
    c = lax.axis_index("c")
    s = lax.axis_index("s")
    wid = s * 2 + c
    tiles_per_b = NTILES // B
    return wid, wid // tiles_per_b, (wid % tiles_per_b) * chunk


def _zero_acc(acc):
    zf = jnp.zeros((LANES,), jnp.float32)
    for l in range(LANES):
        for j in range(NLAB_PAD // LANES):
            acc[l, pl.ds(j * LANES, LANES)] = zf


def _hist_body(hinge_hbm, label_hbm, out_s, out_c,
               hin_v, lab_v, acc_s, acc_c, hsem, lsem):
    chunk = hin_v.shape[0]
    B, M = hinge_hbm.shape
    wid, b, m0 = _tile_coords(chunk, B)

    sub = chunk // NSUB

    def hcopy(j):
        return pltpu.make_async_copy(
            hinge_hbm.at[b, pl.ds(m0 + j * sub, sub)],
            hin_v.at[pl.ds(j * sub, sub)], hsem.at[j])

    def lcopy(j):
        return pltpu.make_async_copy(
            label_hbm.at[b, pl.ds(m0 + j * sub, sub)],
            lab_v.at[pl.ds(j * sub, sub)], lsem.at[j])

    for j in range(NSUB):
        hcopy(j).start()
        lcopy(j).start()

    _zero_acc(acc_s)
    _zero_acc(acc_c)
    lanes = lax.iota(jnp.int32, LANES)
    ones = jnp.ones((LANES,), jnp.float32)

    def body(i, carry):
        labs = [lab_v[pl.ds((i * UNROLL + u) * LANES, LANES)]
                for u in range(UNROLL)]
        hins = [hin_v[pl.ds((i * UNROLL + u) * LANES, LANES)]
                for u in range(UNROLL)]
        for u in range(UNROLL):
            plsc.addupdate_scatter(acc_s, [lanes, labs[u]], hins[u])
        for u in range(UNROLL):
            plsc.addupdate_scatter(acc_c, [lanes, labs[u]], ones)
        return carry

    per_sub = sub // (LANES * UNROLL)
    for j in range(NSUB):
        hcopy(j).wait()
        lcopy(j).wait()
        lax.fori_loop(j * per_sub, (j + 1) * per_sub, body, 0)

    pltpu.sync_copy(acc_s, out_s.at[wid])
    pltpu.sync_copy(acc_c, out_c.at[wid])


def _hist(hinge, label):
    B, M = hinge.shape
    chunk = (B * M) // NTILES
    mesh = plsc.VectorSubcoreMesh(core_axis_name="c", subcore_axis_name="s")
    f32 = jnp.float32
    out_t = (jax.ShapeDtypeStruct((NTILES, LANES, NLAB_PAD), f32),
             jax.ShapeDtypeStruct((NTILES, LANES, NLAB_PAD), f32))
    run = pl.kernel(
        _hist_body,
        out_type=out_t,
        mesh=mesh,
        compiler_params=pltpu.CompilerParams(needs_layout_passes=False),
        scratch_types=[
            pltpu.VMEM((chunk,), f32),
            pltpu.VMEM((chunk,), jnp.int32),
            pltpu.VMEM((LANES, NLAB_PAD), f32),
            pltpu.VMEM((LANES, NLAB_PAD), f32),
            pltpu.SemaphoreType.DMA((NSUB,)),
            pltpu.SemaphoreType.DMA((NSUB,)),
        ],
    )
    return run(hinge, label)


def _combine_body(s_ref, c_ref, o_ref):
    sv = s_ref[...]
    cv = c_ref[...]
    n_b = sv.shape[0] * sv.shape[1] // (4 * LANES)
    s = jnp.sum(sv.reshape(n_b, 4 * LANES, 128), axis=1)
    c = jnp.sum(cv.reshape(n_b, 4 * LANES, 128), axis=1)
    present = c > 0.0
    denom = jnp.where(present, c, 1.0)
    terms = jnp.where(present, s / denom, 0.0)
    inst = jnp.sum(terms, axis=1)
    nu = jnp.sum(present.astype(jnp.float32), axis=1)
    o_ref[...] = jnp.reshape(jnp.sum(inst / nu), (1, 1))


def _combine(sums, counts):
    return pl.pallas_call(
        _combine_body,
        out_shape=jax.ShapeDtypeStruct((1, 1), jnp.float32),
    )(sums, counts)


@jax.jit
def kernel(prediction, label):
    h = _hinge(prediction)
    sums, counts = _hist(h, label)
    out = _combine(sums, counts)
    return out[0, 0]


# speedup vs baseline: 1.0061x; 1.0061x over previous
import functools

import jax
import jax.numpy as jnp
from jax import lax
from jax.experimental import pallas as pl
from jax.experimental.pallas import tpu as pltpu
from jax.experimental.pallas import tpu_sc as plsc

D_VAR_ = 0.5
NLAB_PAD = 128
LANES = 16
NTILES = 32
BM = 8192


NBUF = 20
CHM = 4096


def _hinge_body(x_hbm, o_hbm, bufs, obuf, isem, osem):
    B, C, M = x_hbm.shape
    nchunks = B * (M // CHM)
    per_b = M // CHM

    def in_copy(i, s):
        b, mc = divmod(i, per_b)
        return pltpu.make_async_copy(
            x_hbm.at[b, :, pl.ds(mc * CHM, CHM)], bufs.at[s], isem.at[s])

    def out_copy(i, s):
        b, mc = divmod(i, per_b)
        return pltpu.make_async_copy(
            obuf.at[s], o_hbm.at[pl.ds(b, 1), pl.ds(mc * CHM, CHM)],
            osem.at[s])

    for j in range(NBUF):
        in_copy(j, j).start()

    for i in range(nchunks):
        s = i % NBUF
        in_copy(i, s).wait()
        x = bufs[s]
        s1 = jnp.sum(x, axis=0, keepdims=True)
        s2 = jnp.sum(x * x, axis=0, keepdims=True)
        var = jnp.maximum(s2 - s1 * s1 * (1.0 / C), 0.0)
        h = jnp.maximum(jnp.sqrt(var) - D_VAR_, 0.0)
        if i >= NBUF:
            out_copy(i - NBUF, s).wait()
        obuf[s] = h * h
        out_copy(i, s).start()
        nxt = i + NBUF
        if nxt < nchunks:
            in_copy(nxt, s).start()

    for i in range(nchunks - NBUF, nchunks):
        out_copy(i, i % NBUF).wait()


def _hinge(prediction):
    B, C, M = prediction.shape
    return pl.pallas_call(
        _hinge_body,
        in_specs=[pl.BlockSpec(memory_space=pl.ANY)],
        out_specs=pl.BlockSpec(memory_space=pl.ANY),
        out_shape=jax.ShapeDtypeStruct((B, M), jnp.float32),
        scratch_shapes=[
            pltpu.VMEM((NBUF, 32, CHM), jnp.float32),
            pltpu.VMEM((NBUF, 1, CHM), jnp.float32),
            pltpu.SemaphoreType.DMA((NBUF,)),
            pltpu.SemaphoreType.DMA((NBUF,)),
        ],
    )(prediction)


NSUB = 4
UNROLL = 8


def _tile_coords(chunk, B):
    c = lax.axis_index("c")
    s = lax.axis_index("s")
    wid = s * 2 + c
    tiles_per_b = NTILES // B
    return wid, wid // tiles_per_b, (wid % tiles_per_b) * chunk


def _zero_acc(acc):
    zf = jnp.zeros((LANES,), jnp.float32)
    for l in range(LANES):
        for j in range(NLAB_PAD // LANES):
            acc[l, pl.ds(j * LANES, LANES)] = zf


def _hist_body(hinge_hbm, label_hbm, out_s, out_c,
               hin_v, lab_v, acc_s, acc_c, hsem, lsem):
    chunk = hin_v.shape[0]
    B, M = hinge_hbm.shape
    wid, b, m0 = _tile_coords(chunk, B)

    sub = chunk // NSUB

    def hcopy(j):
        return pltpu.make_async_copy(
            hinge_hbm.at[b, pl.ds(m0 + j * sub, sub)],
            hin_v.at[pl.ds(j * sub, sub)], hsem.at[j])

    def lcopy(j):
        return pltpu.make_async_copy(
            label_hbm.at[b, pl.ds(m0 + j * sub, sub)],
            lab_v.at[pl.ds(j * sub, sub)], lsem.at[j])

    for j in range(NSUB):
        hcopy(j).start()
        lcopy(j).start()

    _zero_acc(acc_s)
    _zero_acc(acc_c)
    lanes = lax.iota(jnp.int32, LANES)
    ones = jnp.ones((LANES,), jnp.float32)

    def body(i, carry):
        labs = [lab_v[pl.ds((i * UNROLL + u) * LANES, LANES)]
                for u in range(UNROLL)]
        hins = [hin_v[pl.ds((i * UNROLL + u) * LANES, LANES)]
                for u in range(UNROLL)]
        for u in range(UNROLL):
            plsc.addupdate_scatter(acc_s, [lanes, labs[u]], hins[u])
        for u in range(UNROLL):
            plsc.addupdate_scatter(acc_c, [lanes, labs[u]], ones)
        return carry

    per_sub = sub // (LANES * UNROLL)
    for j in range(NSUB):
        hcopy(j).wait()
        lcopy(j).wait()
        lax.fori_loop(j * per_sub, (j + 1) * per_sub, body, 0)

    pltpu.sync_copy(acc_s, out_s.at[wid])
    pltpu.sync_copy(acc_c, out_c.at[wid])


def _hist(hinge, label):
    B, M = hinge.shape
    chunk = (B * M) // NTILES
    mesh = plsc.VectorSubcoreMesh(core_axis_name="c", subcore_axis_name="s")
    f32 = jnp.float32
    out_t = (jax.ShapeDtypeStruct((NTILES, LANES, NLAB_PAD), f32),
             jax.ShapeDtypeStruct((NTILES, LANES, NLAB_PAD), f32))
    run = pl.kernel(
        _hist_body,
        out_type=out_t,
        mesh=mesh,
        compiler_params=pltpu.CompilerParams(needs_layout_passes=False),
        scratch_types=[
            pltpu.VMEM((chunk,), f32),
            pltpu.VMEM((chunk,), jnp.int32),
            pltpu.VMEM((LANES, NLAB_PAD), f32),
            pltpu.VMEM((LANES, NLAB_PAD), f32),
            pltpu.SemaphoreType.DMA((NSUB,)),
            pltpu.SemaphoreType.DMA((NSUB,)),
        ],
    )
    return run(hinge, label)


def _combine_body(s_ref, c_ref, o_ref):
    sv = s_ref[...]
    cv = c_ref[...]
    n_b = sv.shape[0] * sv.shape[1] // (4 * LANES)
    s = jnp.sum(sv.reshape(n_b, 4 * LANES, 128), axis=1)
    c = jnp.sum(cv.reshape(n_b, 4 * LANES, 128), axis=1)
    present = c > 0.0
    denom = jnp.where(present, c, 1.0)
    terms = jnp.where(present, s / denom, 0.0)
    inst = jnp.sum(terms, axis=1)
    nu = jnp.sum(present.astype(jnp.float32), axis=1)
    o_ref[...] = jnp.reshape(jnp.sum(inst / nu), (1, 1))


def _combine(sums, counts):
    return pl.pallas_call(
        _combine_body,
        out_shape=jax.ShapeDtypeStruct((1, 1), jnp.float32),
    )(sums, counts)


@jax.jit
def kernel(prediction, label):
    h = _hinge(prediction)
    sums, counts = _hist(h, label)
    out = _combine(sums, counts)
    return out[0, 0]


# speedup vs baseline: 1.0096x; 1.0035x over previous
import functools

import jax
import jax.numpy as jnp
from jax import lax
from jax.experimental import pallas as pl
from jax.experimental.pallas import tpu as pltpu
from jax.experimental.pallas import tpu_sc as plsc

D_VAR_ = 0.5
NLAB_PAD = 128
LANES = 16
NTILES = 32
BM = 8192


NBUF = 16
CHM = 4096


def _hinge_body(x_hbm, o_hbm, bufs, obuf, isem, osem):
    B, C, M = x_hbm.shape
    nchunks = B * (M // CHM)
    per_b = M // CHM

    def in_copy(i, s):
        b, mc = divmod(i, per_b)
        return pltpu.make_async_copy(
            x_hbm.at[b, :, pl.ds(mc * CHM, CHM)], bufs.at[s], isem.at[s])

    def out_copy(i, s):
        b, mc = divmod(i, per_b)
        return pltpu.make_async_copy(
            obuf.at[s], o_hbm.at[pl.ds(b, 1), pl.ds(mc * CHM, CHM)],
            osem.at[s])

    for j in range(NBUF):
        in_copy(j, j).start()

    for i in range(nchunks):
        s = i % NBUF
        in_copy(i, s).wait()
        x = bufs[s]
        s1 = jnp.sum(x, axis=0, keepdims=True)
        s2 = jnp.sum(x * x, axis=0, keepdims=True)
        var = jnp.maximum(s2 - s1 * s1 * (1.0 / C), 0.0)
        h = jnp.maximum(jnp.sqrt(var) - D_VAR_, 0.0)
        if i >= NBUF:
            out_copy(i - NBUF, s).wait()
        obuf[s] = h * h
        out_copy(i, s).start()
        nxt = i + NBUF
        if nxt < nchunks:
            in_copy(nxt, s).start()

    for i in range(nchunks - NBUF, nchunks):
        out_copy(i, i % NBUF).wait()


def _hinge(prediction):
    B, C, M = prediction.shape
    return pl.pallas_call(
        _hinge_body,
        in_specs=[pl.BlockSpec(memory_space=pl.ANY)],
        out_specs=pl.BlockSpec(memory_space=pl.ANY),
        out_shape=jax.ShapeDtypeStruct((B, M), jnp.float32),
        scratch_shapes=[
            pltpu.VMEM((NBUF, 32, CHM), jnp.float32),
            pltpu.VMEM((NBUF, 1, CHM), jnp.float32),
            pltpu.SemaphoreType.DMA((NBUF,)),
            pltpu.SemaphoreType.DMA((NBUF,)),
        ],
    )(prediction)


NSUB = 4
UNROLL = 8


def _tile_coords(chunk, B):
    c = lax.axis_index("c")
    s = lax.axis_index("s")
    wid = s * 2 + c
    tiles_per_b = NTILES // B
    return wid, wid // tiles_per_b, (wid % tiles_per_b) * chunk


def _zero_acc(acc):
    zf = jnp.zeros((LANES,), jnp.float32)
    for l in range(LANES):
        for j in range(NLAB_PAD // LANES):
            acc[l, pl.ds(j * LANES, LANES)] = zf


def _hist_body(hinge_hbm, label_hbm, out_s, out_c,
               hin_v, lab_v, acc_s, acc_c, hsem, lsem):
    chunk = hin_v.shape[0]
    B, M = hinge_hbm.shape
    wid, b, m0 = _tile_coords(chunk, B)

    sub = chunk // NSUB

    def hcopy(j):
        return pltpu.make_async_copy(
            hinge_hbm.at[b, pl.ds(m0 + j * sub, sub)],
            hin_v.at[pl.ds(j * sub, sub)], hsem.at[j])

    def lcopy(j):
        return pltpu.make_async_copy(
            label_hbm.at[b, pl.ds(m0 + j * sub, sub)],
            lab_v.at[pl.ds(j * sub, sub)], lsem.at[j])

    for j in range(NSUB):
        hcopy(j).start()
        lcopy(j).start()

    _zero_acc(acc_s)
    _zero_acc(acc_c)
    lanes = lax.iota(jnp.int32, LANES)
    ones = jnp.ones((LANES,), jnp.float32)

    def body(i, carry):
        labs = [lab_v[pl.ds((i * UNROLL + u) * LANES, LANES)]
                for u in range(UNROLL)]
        hins = [hin_v[pl.ds((i * UNROLL + u) * LANES, LANES)]
                for u in range(UNROLL)]
        for u in range(UNROLL):
            plsc.addupdate_scatter(acc_s, [lanes, labs[u]], hins[u])
        for u in range(UNROLL):
            plsc.addupdate_scatter(acc_c, [lanes, labs[u]], ones)
        return carry

    per_sub = sub // (LANES * UNROLL)
    for j in range(NSUB):
        hcopy(j).wait()
        lcopy(j).wait()
        lax.fori_loop(j * per_sub, (j + 1) * per_sub, body, 0)

    pltpu.sync_copy(acc_s, out_s.at[wid])
    pltpu.sync_copy(acc_c, out_c.at[wid])


def _hist(hinge, label):
    B, M = hinge.shape
    chunk = (B * M) // NTILES
    mesh = plsc.VectorSubcoreMesh(core_axis_name="c", subcore_axis_name="s")
    f32 = jnp.float32
    out_t = (jax.ShapeDtypeStruct((NTILES, LANES, NLAB_PAD), f32),
             jax.ShapeDtypeStruct((NTILES, LANES, NLAB_PAD), f32))
    run = pl.kernel(
        _hist_body,
        out_type=out_t,
        mesh=mesh,
        compiler_params=pltpu.CompilerParams(needs_layout_passes=False),
        scratch_types=[
            pltpu.VMEM((chunk,), f32),
            pltpu.VMEM((chunk,), jnp.int32),
            pltpu.VMEM((LANES, NLAB_PAD), f32),
            pltpu.VMEM((LANES, NLAB_PAD), f32),
            pltpu.SemaphoreType.DMA((NSUB,)),
            pltpu.SemaphoreType.DMA((NSUB,)),
        ],
    )
    return run(hinge, label)


def _combine_body(s_ref, c_ref, o_ref):
    sv = s_ref[...]
    cv = c_ref[...]
    n_b = sv.shape[0] * sv.shape[1] // (4 * LANES)
    s = jnp.sum(sv.reshape(n_b, 4 * LANES, 128), axis=1)
    c = jnp.sum(cv.reshape(n_b, 4 * LANES, 128), axis=1)
    present = c > 0.0
    denom = jnp.where(present, c, 1.0)
    terms = jnp.where(present, s / denom, 0.0)
    inst = jnp.sum(terms, axis=1)
    nu = jnp.sum(present.astype(jnp.float32), axis=1)
    o_ref[...] = jnp.reshape(jnp.sum(inst / nu), (1, 1))


def _combine(sums, counts):
    return pl.pallas_call(
        _combine_body,
        out_shape=jax.ShapeDtypeStruct((1, 1), jnp.float32),
    )(sums, counts)


@jax.jit
def kernel(prediction, label):
    h = _hinge(prediction)
    sums, counts = _hist(h, label)
    out = _combine(sums, counts)
    return out[0, 0]
